# speedup vs baseline: 2.3240x
import functools

import jax
import jax.numpy as jnp
from jax import lax
from jax.experimental import pallas as pl
from jax.experimental.pallas import tpu as pltpu
from jax.experimental.pallas import tpu_sc as plsc

N = 100000
H = 128
MAX_IDX = 511

NC = 2
NS = 16
NW = NC * NS

CHUNK = 80
NUM_CHUNKS = N // CHUNK
ITERS = (NUM_CHUNKS + NW - 1) // NW

_mesh = plsc.VectorSubcoreMesh(core_axis_name="c", subcore_axis_name="s")


@functools.partial(
    pl.kernel,
    out_type=jax.ShapeDtypeStruct((N, H), jnp.float32),
    mesh=_mesh,
    scratch_types=[
        pltpu.VMEM((CHUNK,), jnp.int32),
        pltpu.VMEM((CHUNK,), jnp.int32),
        pltpu.VMEM((CHUNK, H), jnp.float32),
        pltpu.VMEM((CHUNK, H), jnp.float32),
        pltpu.VMEM((CHUNK, H), jnp.float32),
        pltpu.SemaphoreType.DMA,
    ],
)
def _centrality_body(x_hbm, in_deg_hbm, out_deg_hbm, w_in_hbm, w_out_hbm,
                     out_hbm, idx_a, idx_b, acc_v, emb_a, emb_b, sem):
    wid = lax.axis_index("s") * NC + lax.axis_index("c")

    def do_chunk(t, carry):
        ci = t * NW + wid

        @pl.when(ci < NUM_CHUNKS)
        def _():
            base = ci * CHUNK
            rows = pl.ds(base, CHUNK)
            h_x = pltpu.async_copy(x_hbm.at[rows], acc_v, sem)
            pltpu.sync_copy(in_deg_hbm.at[rows], idx_a)
            pltpu.sync_copy(out_deg_hbm.at[rows], idx_b)
            for j in range(CHUNK // 16):
                sl = pl.ds(j * 16, 16)
                idx_a[sl] = jnp.minimum(idx_a[sl], MAX_IDX)
                idx_b[sl] = jnp.minimum(idx_b[sl], MAX_IDX)
            h_a = pltpu.async_copy(w_in_hbm.at[idx_a], emb_a, sem)
            h_b = pltpu.async_copy(w_out_hbm.at[idx_b], emb_b, sem)
            h_x.wait()
            h_a.wait()
            h_b.wait()

            def add_row(r, c2):
                for c in range(H // 16):
                    cs = pl.ds(c * 16, 16)
                    acc_v[r, cs] = acc_v[r, cs] + emb_a[r, cs] + emb_b[r, cs]
                return c2

            lax.fori_loop(0, CHUNK, add_row, 0)
            pltpu.sync_copy(acc_v, out_hbm.at[rows])

        return carry

    lax.fori_loop(0, ITERS, do_chunk, 0)


def kernel(x, in_degree, out_degree, W_in, W_out):
    return _centrality_body(x, in_degree, out_degree, W_in, W_out)


# speedup vs baseline: 3.6171x; 3.6171x over previous
import functools

import jax
import jax.numpy as jnp
from jax import lax
from jax.experimental import pallas as pl
from jax.experimental.pallas import tpu as pltpu
from jax.experimental.pallas import tpu_sc as plsc

N = 100000
H = 128
MAX_IDX = 511

NC = 2
NS = 16
NW = NC * NS

CHUNK = 80
NUM_CHUNKS = N // CHUNK
ITERS = (NUM_CHUNKS + NW - 1) // NW

RING = 3
STEPS = ITERS + RING - 1
ROUNDS = STEPS // RING

_mesh = plsc.VectorSubcoreMesh(core_axis_name="c", subcore_axis_name="s")


@functools.partial(
    pl.kernel,
    out_type=jax.ShapeDtypeStruct((N, H), jnp.float32),
    mesh=_mesh,
    scratch_types=[
        pltpu.VMEM((RING, CHUNK), jnp.int32),
        pltpu.VMEM((RING, CHUNK), jnp.int32),
        pltpu.VMEM((RING, CHUNK, H), jnp.float32),
        [pltpu.SemaphoreType.DMA] * RING,
        [pltpu.SemaphoreType.DMA] * RING,
        [pltpu.SemaphoreType.DMA] * RING,
    ],
)
def _centrality_body(x_hbm, in_deg_hbm, out_deg_hbm, w_in_hbm, w_out_hbm,
                     out_hbm, idx_a, idx_b, acc_v, sems_x, sems_g, sems_st):
    wid = lax.axis_index("s") * NC + lax.axis_index("c")

    def chunk_rows(ci):
        return pl.ds(ci * CHUNK, CHUNK)

    def do_round(o, carry):
        for r in range(RING):
            step = o * RING + r

            ci_l = step * NW + wid

            @pl.when(ci_l < NUM_CHUNKS)
            def _():
                rows = chunk_rows(ci_l)

                @pl.when(step >= RING)
                def _():
                    prev_rows = chunk_rows(ci_l - RING * NW)
                    pltpu.make_async_copy(
                        acc_v.at[r], out_hbm.at[prev_rows], sems_st[r]).wait()

                pltpu.async_copy(x_hbm.at[rows], acc_v.at[r], sems_x[r])
                pltpu.sync_copy(in_deg_hbm.at[rows], idx_a.at[r])
                pltpu.sync_copy(out_deg_hbm.at[rows], idx_b.at[r])
                for j in range(CHUNK // 16):
                    sl = pl.ds(j * 16, 16)
                    idx_a[r, sl] = jnp.minimum(idx_a[r, sl], MAX_IDX)
                    idx_b[r, sl] = jnp.minimum(idx_b[r, sl], MAX_IDX)

            rg = (r - 1) % RING
            t_g = step - 1
            ci_g = t_g * NW + wid

            @pl.when(jnp.logical_and(t_g >= 0, ci_g < NUM_CHUNKS))
            def _():
                rows = chunk_rows(ci_g)
                pltpu.make_async_copy(
                    x_hbm.at[rows], acc_v.at[rg], sems_x[rg]).wait()
                pltpu.async_copy(
                    w_in_hbm.at[idx_a.at[rg]], acc_v.at[rg], sems_g[rg],
                    add=True)
                pltpu.async_copy(
                    w_out_hbm.at[idx_b.at[rg]], acc_v.at[rg], sems_g[rg],
                    add=True)

            rs = (r - 2) % RING
            t_s = step - 2
            ci_s = t_s * NW + wid

            @pl.when(jnp.logical_and(t_s >= 0, ci_s < NUM_CHUNKS))
            def _():
                rows = chunk_rows(ci_s)
                gather_done = pltpu.make_async_copy(
                    x_hbm.at[rows], acc_v.at[rs], sems_g[rs])
                gather_done.wait()
                gather_done.wait()
                pltpu.async_copy(acc_v.at[rs], out_hbm.at[rows], sems_st[rs])

        return carry

    lax.fori_loop(0, ROUNDS, do_round, 0)

    for t in range(max(ITERS - RING - 1, 0), ITERS):
        ci = t * NW + wid

        @pl.when(jnp.logical_and(ci < NUM_CHUNKS,
                                 ci + RING * NW >= NUM_CHUNKS))
        def _():
            pltpu.make_async_copy(
                acc_v.at[t % RING], out_hbm.at[chunk_rows(ci)],
                sems_st[t % RING]).wait()


def kernel(x, in_degree, out_degree, W_in, W_out):
    return _centrality_body(x, in_degree, out_degree, W_in, W_out)


# speedup vs baseline: 3.7338x; 1.0322x over previous
import functools

import jax
import jax.numpy as jnp
from jax import lax
from jax.experimental import pallas as pl
from jax.experimental.pallas import tpu as pltpu
from jax.experimental.pallas import tpu_sc as plsc

N = 100000
H = 128
MAX_IDX = 511

NC = 2
NS = 16
NW = NC * NS

CHUNK = 160
SUB = 80
NUM_CHUNKS = N // CHUNK
ITERS = (NUM_CHUNKS + NW - 1) // NW

RING = 3
STEPS = ITERS + RING - 1
ROUNDS = (STEPS + RING - 1) // RING

_mesh = plsc.VectorSubcoreMesh(core_axis_name="c", subcore_axis_name="s")


@functools.partial(
    pl.kernel,
    out_type=jax.ShapeDtypeStruct((N, H), jnp.float32),
    mesh=_mesh,
    scratch_types=[
        pltpu.VMEM((RING, CHUNK // SUB, SUB), jnp.int32),
        pltpu.VMEM((RING, CHUNK // SUB, SUB), jnp.int32),
        pltpu.VMEM((RING, CHUNK, H), jnp.float32),
        [pltpu.SemaphoreType.DMA] * RING,
        [pltpu.SemaphoreType.DMA] * RING,
        [pltpu.SemaphoreType.DMA] * RING,
        [pltpu.SemaphoreType.DMA] * RING,
    ],
)
def _centrality_body(x_hbm, in_deg_hbm, out_deg_hbm, w_in_hbm, w_out_hbm,
                     out_hbm, idx_a, idx_b, acc_v, sems_x, sems_i, sems_g,
                     sems_st):
    wid = lax.axis_index("s") * NC + lax.axis_index("c")

    def chunk_rows(ci):
        return pl.ds(ci * CHUNK, CHUNK)

    def do_round(o, carry):
        for r in range(RING):
            step = o * RING + r

            ci_l = step * NW + wid

            @pl.when(ci_l < NUM_CHUNKS)
            def _():
                rows = chunk_rows(ci_l)

                @pl.when(step >= RING)
                def _():
                    prev_rows = chunk_rows(ci_l - RING * NW)
                    pltpu.make_async_copy(
                        acc_v.at[r], out_hbm.at[prev_rows], sems_st[r]).wait()

                pltpu.async_copy(x_hbm.at[rows], acc_v.at[r], sems_x[r])
                for g in range(CHUNK // SUB):
                    sub = pl.ds(ci_l * CHUNK + g * SUB, SUB)
                    pltpu.async_copy(in_deg_hbm.at[sub], idx_a.at[r, g],
                                     sems_i[r])
                    pltpu.async_copy(out_deg_hbm.at[sub], idx_b.at[r, g],
                                     sems_i[r])

            rg = (r - 1) % RING
            t_g = step - 1
            ci_g = t_g * NW + wid

            @pl.when(jnp.logical_and(t_g >= 0, ci_g < NUM_CHUNKS))
            def _():
                rows = chunk_rows(ci_g)
                idx_done = pltpu.make_async_copy(
                    in_deg_hbm.at[pl.ds(ci_g * CHUNK, SUB)],
                    idx_a.at[rg, 0], sems_i[rg])
                for _c in range(2 * (CHUNK // SUB)):
                    idx_done.wait()
                for g in range(CHUNK // SUB):
                    for j in range(SUB // 16):
                        sl = pl.ds(j * 16, 16)
                        idx_a[rg, g, sl] = jnp.minimum(idx_a[rg, g, sl],
                                                       MAX_IDX)
                        idx_b[rg, g, sl] = jnp.minimum(idx_b[rg, g, sl],
                                                       MAX_IDX)
                pltpu.make_async_copy(
                    x_hbm.at[rows], acc_v.at[rg], sems_x[rg]).wait()
                for g in range(CHUNK // SUB):
                    rsl = pl.ds(g * SUB, SUB)
                    pltpu.async_copy(
                        w_in_hbm.at[idx_a.at[rg, g]], acc_v.at[rg, rsl],
                        sems_g[rg], add=True)
                    pltpu.async_copy(
                        w_out_hbm.at[idx_b.at[rg, g]], acc_v.at[rg, rsl],
                        sems_g[rg], add=True)

            rs = (r - 2) % RING
            t_s = step - 2
            ci_s = t_s * NW + wid

            @pl.when(jnp.logical_and(t_s >= 0, ci_s < NUM_CHUNKS))
            def _():
                rows = chunk_rows(ci_s)
                gather_done = pltpu.make_async_copy(
                    x_hbm.at[pl.ds(ci_s * CHUNK, SUB)],
                    acc_v.at[rs, pl.ds(0, SUB)], sems_g[rs])
                for _g in range(2 * (CHUNK // SUB)):
                    gather_done.wait()
                pltpu.async_copy(acc_v.at[rs], out_hbm.at[rows], sems_st[rs])

        return carry

    lax.fori_loop(0, ROUNDS, do_round, 0)

    for t in range(max(ITERS - RING - 1, 0), ITERS):
        ci = t * NW + wid

        @pl.when(jnp.logical_and(ci < NUM_CHUNKS,
                                 ci + RING * NW >= NUM_CHUNKS))
        def _():
            pltpu.make_async_copy(
                acc_v.at[t % RING], out_hbm.at[chunk_rows(ci)],
                sems_st[t % RING]).wait()


def kernel(x, in_degree, out_degree, W_in, W_out):
    return _centrality_body(x, in_degree, out_degree, W_in, W_out)


# speedup vs baseline: 6.1146x; 1.6376x over previous
import functools

import jax
import jax.numpy as jnp
from jax import lax
from jax.experimental import pallas as pl
from jax.experimental.pallas import tpu as pltpu
from jax.experimental.pallas import tpu_sc as plsc

N = 100000
H = 128
MAX_IDX = 511

NC = 2
NS = 16
NW = NC * NS

CHUNK = 160
SUB = 80
NUM_CHUNKS = N // CHUNK
ITERS = (NUM_CHUNKS + NW - 1) // NW

RING = 3
STEPS = ITERS + RING - 1
ROUNDS = (STEPS + RING - 1) // RING

_mesh = plsc.VectorSubcoreMesh(core_axis_name="c", subcore_axis_name="s")


@functools.partial(
    pl.kernel,
    out_type=jax.ShapeDtypeStruct((N, H), jnp.float32),
    mesh=_mesh,
    scratch_types=[
        pltpu.VMEM((RING, CHUNK // SUB, SUB), jnp.int32),
        pltpu.VMEM((RING, CHUNK // SUB, SUB), jnp.int32),
        pltpu.VMEM((RING, CHUNK, H), jnp.float32),
        pltpu.VMEM_SHARED((512, H), jnp.float32),
        pltpu.VMEM_SHARED((512, H), jnp.float32),
        [pltpu.SemaphoreType.DMA] * RING,
        [pltpu.SemaphoreType.DMA] * RING,
        [pltpu.SemaphoreType.DMA] * RING,
        [pltpu.SemaphoreType.DMA] * RING,
    ],
)
def _centrality_body(x_hbm, in_deg_hbm, out_deg_hbm, w_in_hbm, w_out_hbm,
                     out_hbm, idx_a, idx_b, acc_v, w_in_sh, w_out_sh,
                     sems_x, sems_i, sems_g, sems_st):
    wid = lax.axis_index("s") * NC + lax.axis_index("c")

    @pl.when(lax.axis_index("s") == 0)
    def _():
        pltpu.sync_copy(w_in_hbm.at[pl.ds(0, 512)], w_in_sh)
        pltpu.sync_copy(w_out_hbm.at[pl.ds(0, 512)], w_out_sh)

    plsc.subcore_barrier()

    def chunk_rows(ci):
        return pl.ds(ci * CHUNK, CHUNK)

    def do_round(o, carry):
        for r in range(RING):
            step = o * RING + r

            ci_l = step * NW + wid

            @pl.when(ci_l < NUM_CHUNKS)
            def _():
                rows = chunk_rows(ci_l)

                @pl.when(step >= RING)
                def _():
                    prev_rows = chunk_rows(ci_l - RING * NW)
                    pltpu.make_async_copy(
                        acc_v.at[r], out_hbm.at[prev_rows], sems_st[r]).wait()

                pltpu.async_copy(x_hbm.at[rows], acc_v.at[r], sems_x[r])
                for g in range(CHUNK // SUB):
                    sub = pl.ds(ci_l * CHUNK + g * SUB, SUB)
                    pltpu.async_copy(in_deg_hbm.at[sub], idx_a.at[r, g],
                                     sems_i[r])
                    pltpu.async_copy(out_deg_hbm.at[sub], idx_b.at[r, g],
                                     sems_i[r])

            rg = (r - 1) % RING
            t_g = step - 1
            ci_g = t_g * NW + wid

            @pl.when(jnp.logical_and(t_g >= 0, ci_g < NUM_CHUNKS))
            def _():
                rows = chunk_rows(ci_g)
                idx_done = pltpu.make_async_copy(
                    in_deg_hbm.at[pl.ds(ci_g * CHUNK, SUB)],
                    idx_a.at[rg, 0], sems_i[rg])
                for _c in range(2 * (CHUNK // SUB)):
                    idx_done.wait()
                for g in range(CHUNK // SUB):
                    for j in range(SUB // 16):
                        sl = pl.ds(j * 16, 16)
                        idx_a[rg, g, sl] = jnp.minimum(idx_a[rg, g, sl],
                                                       MAX_IDX)
                        idx_b[rg, g, sl] = jnp.minimum(idx_b[rg, g, sl],
                                                       MAX_IDX)
                pltpu.make_async_copy(
                    x_hbm.at[rows], acc_v.at[rg], sems_x[rg]).wait()
                for g in range(CHUNK // SUB):
                    rsl = pl.ds(g * SUB, SUB)
                    pltpu.async_copy(
                        w_in_sh.at[idx_a.at[rg, g]], acc_v.at[rg, rsl],
                        sems_g[rg], add=True)
                    pltpu.async_copy(
                        w_out_sh.at[idx_b.at[rg, g]], acc_v.at[rg, rsl],
                        sems_g[rg], add=True)

            rs = (r - 2) % RING
            t_s = step - 2
            ci_s = t_s * NW + wid

            @pl.when(jnp.logical_and(t_s >= 0, ci_s < NUM_CHUNKS))
            def _():
                rows = chunk_rows(ci_s)
                gather_done = pltpu.make_async_copy(
                    x_hbm.at[pl.ds(ci_s * CHUNK, SUB)],
                    acc_v.at[rs, pl.ds(0, SUB)], sems_g[rs])
                for _g in range(2 * (CHUNK // SUB)):
                    gather_done.wait()
                pltpu.async_copy(acc_v.at[rs], out_hbm.at[rows], sems_st[rs])

        return carry

    lax.fori_loop(0, ROUNDS, do_round, 0)

    for t in range(max(ITERS - RING - 1, 0), ITERS):
        ci = t * NW + wid

        @pl.when(jnp.logical_and(ci < NUM_CHUNKS,
                                 ci + RING * NW >= NUM_CHUNKS))
        def _():
            pltpu.make_async_copy(
                acc_v.at[t % RING], out_hbm.at[chunk_rows(ci)],
                sems_st[t % RING]).wait()


def kernel(x, in_degree, out_degree, W_in, W_out):
    return _centrality_body(x, in_degree, out_degree, W_in, W_out)


# speedup vs baseline: 6.3249x; 1.0344x over previous
import functools

import jax
import jax.numpy as jnp
from jax import lax
from jax.experimental import pallas as pl
from jax.experimental.pallas import tpu as pltpu
from jax.experimental.pallas import tpu_sc as plsc

N = 100000
H = 128
MAX_IDX = 511

NC = 2
NS = 16
NW = NC * NS

CHUNK = 160
SUB = 80
NSUB = CHUNK // SUB
NUM_CHUNKS = N // CHUNK
ITERS = (NUM_CHUNKS + NW - 1) // NW

RING = 5
OFF_G = 1
OFF_S = 3
STEPS = ITERS + OFF_S
ROUNDS = (STEPS + RING - 1) // RING

_mesh = plsc.VectorSubcoreMesh(core_axis_name="c", subcore_axis_name="s")


@functools.partial(
    pl.kernel,
    out_type=jax.ShapeDtypeStruct((N, H), jnp.float32),
    mesh=_mesh,
    scratch_types=[
        pltpu.VMEM((RING, NSUB, SUB), jnp.int32),
        pltpu.VMEM((RING, NSUB, SUB), jnp.int32),
        pltpu.VMEM((RING, CHUNK, H), jnp.float32),
        pltpu.VMEM_SHARED((512, H), jnp.float32),
        pltpu.VMEM_SHARED((512, H), jnp.float32),
        [pltpu.SemaphoreType.DMA] * RING,
        [pltpu.SemaphoreType.DMA] * RING,
        [pltpu.SemaphoreType.DMA] * RING,
        [pltpu.SemaphoreType.DMA] * RING,
    ],
)
def _centrality_body(x_hbm, in_deg_hbm, out_deg_hbm, w_in_hbm, w_out_hbm,
                     out_hbm, idx_a, idx_b, acc_v, w_in_sh, w_out_sh,
                     sems_x, sems_i, sems_g, sems_st):
    wid = lax.axis_index("s") * NC + lax.axis_index("c")

    @pl.when(lax.axis_index("s") == 0)
    def _():
        pltpu.sync_copy(w_in_hbm.at[pl.ds(0, 512)], w_in_sh)
        pltpu.sync_copy(w_out_hbm.at[pl.ds(0, 512)], w_out_sh)

    plsc.subcore_barrier()

    def chunk_rows(ci):
        return pl.ds(ci * CHUNK, CHUNK)

    def do_round(o, carry):
        for r in range(RING):
            step = o * RING + r

            ci_l = step * NW + wid

            @pl.when(ci_l < NUM_CHUNKS)
            def _():
                rows = chunk_rows(ci_l)

                @pl.when(step >= RING)
                def _():
                    prev_rows = chunk_rows(ci_l - RING * NW)
                    pltpu.make_async_copy(
                        acc_v.at[r], out_hbm.at[prev_rows], sems_st[r]).wait()

                pltpu.async_copy(x_hbm.at[rows], acc_v.at[r], sems_x[r])
                for g in range(NSUB):
                    sub = pl.ds(ci_l * CHUNK + g * SUB, SUB)
                    pltpu.async_copy(in_deg_hbm.at[sub], idx_a.at[r, g],
                                     sems_i[r])
                    pltpu.async_copy(out_deg_hbm.at[sub], idx_b.at[r, g],
                                     sems_i[r])

            rg = (r - OFF_G) % RING
            t_g = step - OFF_G
            ci_g = t_g * NW + wid

            @pl.when(jnp.logical_and(t_g >= 0, ci_g < NUM_CHUNKS))
            def _():
                rows = chunk_rows(ci_g)
                idx_done = pltpu.make_async_copy(
                    in_deg_hbm.at[pl.ds(ci_g * CHUNK, SUB)],
                    idx_a.at[rg, 0], sems_i[rg])
                for _c in range(2 * NSUB):
                    idx_done.wait()
                for g in range(NSUB):
                    for j in range(SUB // 16):
                        sl = pl.ds(j * 16, 16)
                        idx_a[rg, g, sl] = jnp.minimum(idx_a[rg, g, sl],
                                                       MAX_IDX)
                        idx_b[rg, g, sl] = jnp.minimum(idx_b[rg, g, sl],
                                                       MAX_IDX)
                pltpu.make_async_copy(
                    x_hbm.at[rows], acc_v.at[rg], sems_x[rg]).wait()
                for g in range(NSUB):
                    rsl = pl.ds(g * SUB, SUB)
                    pltpu.async_copy(
                        w_in_sh.at[idx_a.at[rg, g]], acc_v.at[rg, rsl],
                        sems_g[rg], add=True)
                    pltpu.async_copy(
                        w_out_sh.at[idx_b.at[rg, g]], acc_v.at[rg, rsl],
                        sems_g[rg], add=True)

            rs = (r - OFF_S) % RING
            t_s = step - OFF_S
            ci_s = t_s * NW + wid

            @pl.when(jnp.logical_and(t_s >= 0, ci_s < NUM_CHUNKS))
            def _():
                rows = chunk_rows(ci_s)
                gather_done = pltpu.make_async_copy(
                    x_hbm.at[pl.ds(ci_s * CHUNK, SUB)],
                    acc_v.at[rs, pl.ds(0, SUB)], sems_g[rs])
                for _g in range(2 * NSUB):
                    gather_done.wait()
                pltpu.async_copy(acc_v.at[rs], out_hbm.at[rows], sems_st[rs])

        return carry

    lax.fori_loop(0, ROUNDS, do_round, 0)

    for t in range(max(ITERS - RING - 1, 0), ITERS):
        ci = t * NW + wid

        @pl.when(jnp.logical_and(ci < NUM_CHUNKS,
                                 ci + RING * NW >= NUM_CHUNKS))
        def _():
            pltpu.make_async_copy(
                acc_v.at[t % RING], out_hbm.at[chunk_rows(ci)],
                sems_st[t % RING]).wait()


def kernel(x, in_degree, out_degree, W_in, W_out):
    return _centrality_body(x, in_degree, out_degree, W_in, W_out)
